# Initial kernel scaffold; baseline (speedup 1.0000x reference)
import jax
import jax.numpy as jnp
from jax.experimental import pallas as pl


def kernel(obs, enc_w, enc_b, enc_g, enc_bt, tem_w, tem_b, g0_w, g0_as, g0_ad, g0_b, g0_g, g0_bt, g1_w, g1_as, g1_ad, g1_b, g1_g, g1_bt, act_w1, act_b1, act_w2, act_b2, cr_w1, cr_b1, cr_w2, cr_b2):
    raise NotImplementedError("write your pallas kernel here")


# speedup vs baseline: 41.2296x; 41.2296x over previous
import functools

import jax
import jax.numpy as jnp
from jax import lax
from jax.experimental import pallas as pl
from jax.experimental.pallas import tpu as pltpu

H = 4
C = 32
HID = 128
K = 10
TILE = 256
NEG = -1e30

_DN = (((0,), (0,)), ((), ()))


def _tdot(a, b, precision=None):
    return lax.dot_general(a, b, _DN, precision=precision,
                           preferred_element_type=jnp.float32)


def _enc_kernel(obs_ref, encw_ref, encb_ref, encg_ref, encbt_ref,
                temw_ref, temb_ref, w0_ref, as0_ref, ad0_ref,
                xw_ref, asad_ref):
    blk = obs_ref[...]
    phys = blk[:, :9]
    lat = blk[:, 9:10]
    e = jnp.maximum(
        jnp.dot(phys, encw_ref[...], preferred_element_type=jnp.float32)
        + encb_ref[...], 0.0)
    mu = jnp.mean(e, axis=1, keepdims=True)
    var = jnp.mean((e - mu) * (e - mu), axis=1, keepdims=True)
    x = (e - mu) / jnp.sqrt(var + 1e-5) * encg_ref[...] + encbt_ref[...]
    t = lat * temw_ref[...] + temb_ref[...]
    x = x + t * (1.0 / (1.0 + jnp.exp(-t)))
    xw = jnp.dot(x, w0_ref[...], preferred_element_type=jnp.float32)
    a_s = jnp.dot(xw, as0_ref[...], preferred_element_type=jnp.float32)
    a_d = jnp.dot(xw, ad0_ref[...], preferred_element_type=jnp.float32)
    xw_ref[...] = xw
    asad_ref[...] = jnp.concatenate([a_s, a_d], axis=1)


def _knn_kernel(post_ref, posa_ref, p2r_ref, p2c_ref, out_ref, *, n):
    pos_t = post_ref[0]
    pos_a = posa_ref[0]
    cross = _tdot(pos_t, pos_a)
    d2 = p2c_ref[0] + p2r_ref[0] - 2.0 * cross
    dist = jnp.sqrt(jnp.maximum(d2, 0.0))
    t = pl.program_id(1)
    rows = t * TILE + lax.broadcasted_iota(jnp.int32, (TILE, 1), 0)
    cols = lax.broadcasted_iota(jnp.int32, (TILE, n), 1)
    dist = jnp.where(cols == rows, dist + 1e10, dist)
    sel = jnp.full((TILE, 16), n + 1, jnp.int32)
    hdr = lax.broadcasted_iota(jnp.int32, (TILE, 16), 1)
    for k in range(K):
        mval = jnp.min(dist, axis=1, keepdims=True)
        cand = jnp.where(dist == mval, cols, n)
        idx = jnp.min(cand, axis=1, keepdims=True)
        sel = jnp.where(hdr == k, idx, sel)
        dist = jnp.where(cols == idx, jnp.float32(jnp.inf), dist)
    out_ref[0] = sel


def _flash_step(xw_ref, as_ref, adt_ref, knn_ref, m_scr, l_scr, acc_scr, *, n):
    t = pl.program_id(1)

    @pl.when(t == 0)
    def _init():
        m_scr[...] = jnp.full((8, n), NEG, jnp.float32)
        l_scr[...] = jnp.zeros((8, n), jnp.float32)
        acc_scr[...] = jnp.zeros((HID, n), jnp.float32)

    knn = knn_ref[0]
    cols = lax.broadcasted_iota(jnp.int32, (TILE, n), 1)
    rows = t * TILE + lax.broadcasted_iota(jnp.int32, (TILE, 1), 0)
    mask = cols == rows
    for k in range(K):
        mask = mask | (cols == knn[:, k:k + 1])
    xw = xw_ref[...]
    ad_all = adt_ref[0]
    for h in range(H):
        a = as_ref[:, h:h + 1] + ad_all[h:h + 1, :]
        a = jnp.where(a >= 0, a, 0.2 * a)
        a = jnp.where(mask, a, NEG)
        m_old = m_scr[h:h + 1, :]
        m_new = jnp.maximum(m_old, jnp.max(a, axis=0, keepdims=True))
        corr = jnp.exp(m_old - m_new)
        e = jnp.exp(a - m_new)
        l_scr[h:h + 1, :] = l_scr[h:h + 1, :] * corr + jnp.sum(
            e, axis=0, keepdims=True)
        m_scr[h:h + 1, :] = m_new
        sl = slice(C * h, C * (h + 1))
        acc_scr[sl, :] = acc_scr[sl, :] * corr + _tdot(xw[:, sl], e)


def _ln_relu(y, g, b):
    mu = jnp.mean(y, axis=0, keepdims=True)
    var = jnp.mean((y - mu) * (y - mu), axis=0, keepdims=True)
    return jnp.maximum((y - mu) / jnp.sqrt(var + 1e-5) * g + b, 0.0)


def _gat_out(acc_scr, l_scr, bias_ref, g_ref, bt_ref, *, n):
    l_full = jnp.concatenate(
        [jnp.broadcast_to(l_scr[h:h + 1, :], (C, n)) for h in range(H)],
        axis=0)
    y = acc_scr[...] / (l_full + 1e-16) + bias_ref[...]
    return _ln_relu(y, g_ref[...], bt_ref[...])


def _gat0_kernel(xw_ref, as_ref, adt_ref, knn_ref, bias_ref, g_ref, bt_ref,
                 w1_ref, as1_ref, ad1_ref, xw1t_ref, asad1_ref,
                 m_scr, l_scr, acc_scr, *, n, nt):
    _flash_step(xw_ref, as_ref, adt_ref, knn_ref, m_scr, l_scr, acc_scr, n=n)

    @pl.when(pl.program_id(1) == nt - 1)
    def _fin():
        x1 = _gat_out(acc_scr, l_scr, bias_ref, g_ref, bt_ref, n=n)
        xw1t = _tdot(w1_ref[...], x1)
        as1t = _tdot(as1_ref[...], xw1t)
        ad1t = _tdot(ad1_ref[...], xw1t)
        xw1t_ref[0] = xw1t
        asad1_ref[0] = jnp.concatenate([as1t, ad1t], axis=0)


def _gat1_kernel(xw_ref, as_ref, adt_ref, knn_ref, bias_ref, g_ref, bt_ref,
                 aw1_ref, ab1_ref, aw2_ref, ab2_ref,
                 cw1_ref, cb1_ref, cw2_ref, cb2_ref, head_ref,
                 m_scr, l_scr, acc_scr, *, n, nt):
    _flash_step(xw_ref, as_ref, adt_ref, knn_ref, m_scr, l_scr, acc_scr, n=n)

    @pl.when(pl.program_id(1) == nt - 1)
    def _fin():
        x2 = _gat_out(acc_scr, l_scr, bias_ref, g_ref, bt_ref, n=n)
        h_a = jnp.maximum(_tdot(aw1_ref[...], x2) + ab1_ref[...], 0.0)
        act = jnp.tanh(_tdot(aw2_ref[...], h_a) + ab2_ref[...])
        h_c = jnp.maximum(_tdot(cw1_ref[...], x2) + cb1_ref[...], 0.0)
        val = _tdot(cw2_ref[...], h_c) + cb2_ref[...]
        riota = lax.broadcasted_iota(jnp.int32, (8, n), 0)
        head_ref[0] = jnp.where(riota == 3, val, act)


def _mix(a):
    return (jnp.eye(H, dtype=a.dtype)[:, None, :] * a[:, :, None]).reshape(
        H * C, H)


@jax.jit
def kernel(obs, enc_w, enc_b, enc_g, enc_bt, tem_w, tem_b,
           g0_w, g0_as, g0_ad, g0_b, g0_g, g0_bt,
           g1_w, g1_as, g1_ad, g1_b, g1_g, g1_bt,
           act_w1, act_b1, act_w2, act_b2, cr_w1, cr_b1, cr_w2, cr_b2):
    B, N, D = obs.shape
    Nt = B * N
    NT = N // TILE
    NB = Nt // TILE
    f32 = jnp.float32

    flat = obs.reshape(Nt, D)
    pos_t = jnp.transpose(obs[:, :, :3], (0, 2, 1))
    p2 = jnp.sum(pos_t * pos_t, axis=1)
    p2r = p2[:, None, :]
    p2c = p2[:, :, None]

    row2 = lambda v: v.reshape(1, -1)
    col2 = lambda v: v.reshape(-1, 1)
    full = lambda a: pl.BlockSpec(a.shape, lambda b, t: (0,) * a.ndim)

    As0, Ad0 = _mix(g0_as), _mix(g0_ad)
    xw0, asad0 = pl.pallas_call(
        _enc_kernel,
        grid=(NB,),
        in_specs=[
            pl.BlockSpec((TILE, D), lambda i: (i, 0)),
            pl.BlockSpec((9, HID), lambda i: (0, 0)),
            pl.BlockSpec((1, HID), lambda i: (0, 0)),
            pl.BlockSpec((1, HID), lambda i: (0, 0)),
            pl.BlockSpec((1, HID), lambda i: (0, 0)),
            pl.BlockSpec((1, HID), lambda i: (0, 0)),
            pl.BlockSpec((1, HID), lambda i: (0, 0)),
            pl.BlockSpec((HID, HID), lambda i: (0, 0)),
            pl.BlockSpec((HID, H), lambda i: (0, 0)),
            pl.BlockSpec((HID, H), lambda i: (0, 0)),
        ],
        out_specs=[
            pl.BlockSpec((TILE, HID), lambda i: (i, 0)),
            pl.BlockSpec((TILE, 2 * H), lambda i: (i, 0)),
        ],
        out_shape=[
            jax.ShapeDtypeStruct((Nt, HID), f32),
            jax.ShapeDtypeStruct((Nt, 2 * H), f32),
        ],
    )(flat, enc_w, row2(enc_b), row2(enc_g), row2(enc_bt),
      row2(tem_w), row2(tem_b), g0_w, As0, Ad0)

    knn = pl.pallas_call(
        functools.partial(_knn_kernel, n=N),
        grid=(B, NT),
        in_specs=[
            pl.BlockSpec((1, 3, TILE), lambda b, t: (b, 0, t)),
            pl.BlockSpec((1, 3, N), lambda b, t: (b, 0, 0)),
            pl.BlockSpec((1, 1, N), lambda b, t: (b, 0, 0)),
            pl.BlockSpec((1, TILE, 1), lambda b, t: (b, t, 0)),
        ],
        out_specs=pl.BlockSpec((1, TILE, 16), lambda b, t: (b, t, 0)),
        out_shape=jax.ShapeDtypeStruct((B, N, 16), jnp.int32),
    )(pos_t, pos_t, p2r, p2c)

    scratch = [
        pltpu.VMEM((8, N), f32),
        pltpu.VMEM((8, N), f32),
        pltpu.VMEM((HID, N), f32),
    ]
    flash_specs = [
        pl.BlockSpec((TILE, HID), lambda b, t: (b * NT + t, 0)),
        pl.BlockSpec((TILE, H), lambda b, t: (b * NT + t, 0)),
        pl.BlockSpec((1, H, N), lambda b, t: (b, 0, 0)),
        pl.BlockSpec((1, TILE, 16), lambda b, t: (b, t, 0)),
    ]

    As1, Ad1 = _mix(g1_as), _mix(g1_ad)
    as0 = asad0[:, :H]
    ad0t = jnp.transpose(asad0[:, H:].reshape(B, N, H), (0, 2, 1))
    xw1t, asad1t = pl.pallas_call(
        functools.partial(_gat0_kernel, n=N, nt=NT),
        grid=(B, NT),
        in_specs=flash_specs + [
            full(col2(g0_b)), full(col2(g0_g)), full(col2(g0_bt)),
            full(g1_w), full(As1), full(Ad1),
        ],
        out_specs=[
            pl.BlockSpec((1, HID, N), lambda b, t: (b, 0, 0)),
            pl.BlockSpec((1, 2 * H, N), lambda b, t: (b, 0, 0)),
        ],
        out_shape=[
            jax.ShapeDtypeStruct((B, HID, N), f32),
            jax.ShapeDtypeStruct((B, 2 * H, N), f32),
        ],
        scratch_shapes=scratch,
    )(xw0, as0, ad0t, knn, col2(g0_b), col2(g0_g), col2(g0_bt),
      g1_w, As1, Ad1)

    xw1 = jnp.transpose(xw1t, (0, 2, 1)).reshape(Nt, HID)
    as1 = jnp.transpose(asad1t[:, :H, :], (0, 2, 1)).reshape(Nt, H)
    ad1t = asad1t[:, H:, :]
    aw2 = jnp.pad(act_w2, ((0, 0), (0, 8 - act_w2.shape[1])))
    ab2 = col2(jnp.pad(act_b2, (0, 8 - act_b2.shape[0])))
    cw2 = jnp.pad(cr_w2, ((0, 0), (3, 4)))
    cb2 = col2(jnp.pad(cr_b2, (3, 4)))
    head = pl.pallas_call(
        functools.partial(_gat1_kernel, n=N, nt=NT),
        grid=(B, NT),
        in_specs=flash_specs + [
            full(col2(g1_b)), full(col2(g1_g)), full(col2(g1_bt)),
            full(act_w1), full(col2(act_b1)), full(aw2), full(ab2),
            full(cr_w1), full(col2(cr_b1)), full(cw2), full(cb2),
        ],
        out_specs=pl.BlockSpec((1, 8, N), lambda b, t: (b, 0, 0)),
        out_shape=jax.ShapeDtypeStruct((B, 8, N), f32),
        scratch_shapes=scratch,
    )(xw1, as1, ad1t, knn, col2(g1_b), col2(g1_g), col2(g1_bt),
      act_w1, col2(act_b1), aw2, ab2, cr_w1, col2(cr_b1), cw2, cb2)

    action = jnp.transpose(head[:, :3, :], (0, 2, 1))
    value = jnp.transpose(head[:, 3:4, :], (0, 2, 1))
    return (action, value)


# speedup vs baseline: 47.4697x; 1.1514x over previous
import functools

import jax
import jax.numpy as jnp
from jax import lax
from jax.experimental import pallas as pl
from jax.experimental.pallas import tpu as pltpu

H = 4
C = 32
HID = 128
K = 10
TILE = 256
NEG = -1e30

_DN = (((0,), (0,)), ((), ()))


def _tdot(a, b, precision=None):
    return lax.dot_general(a, b, _DN, precision=precision,
                           preferred_element_type=jnp.float32)


def _enc_kernel(obs_ref, encw_ref, encb_ref, encg_ref, encbt_ref,
                temw_ref, temb_ref, w0_ref, as0_ref, ad0_ref,
                xw_ref, asad_ref):
    blk = obs_ref[...]
    phys = blk[:, :9]
    lat = blk[:, 9:10]
    e = jnp.maximum(
        jnp.dot(phys, encw_ref[...], preferred_element_type=jnp.float32)
        + encb_ref[...], 0.0)
    mu = jnp.mean(e, axis=1, keepdims=True)
    var = jnp.mean((e - mu) * (e - mu), axis=1, keepdims=True)
    x = (e - mu) / jnp.sqrt(var + 1e-5) * encg_ref[...] + encbt_ref[...]
    t = lat * temw_ref[...] + temb_ref[...]
    x = x + t * (1.0 / (1.0 + jnp.exp(-t)))
    xw = jnp.dot(x, w0_ref[...], preferred_element_type=jnp.float32)
    a_s = jnp.dot(xw, as0_ref[...], preferred_element_type=jnp.float32)
    a_d = jnp.dot(xw, ad0_ref[...], preferred_element_type=jnp.float32)
    xw_ref[...] = xw
    asad_ref[...] = jnp.concatenate([a_s, a_d], axis=1)


def _knn_kernel(post_ref, posa_ref, p2r_ref, p2c_ref, out_ref, *, n):
    pos_t = post_ref[0]
    pos_a = posa_ref[0]
    cross = _tdot(pos_t, pos_a)
    d2 = p2c_ref[0] + p2r_ref[0] - 2.0 * cross
    dist = jnp.sqrt(jnp.maximum(d2, 0.0))
    t = pl.program_id(1)
    rows = t * TILE + lax.broadcasted_iota(jnp.int32, (TILE, 1), 0)
    cols = lax.broadcasted_iota(jnp.int32, (TILE, n), 1)
    self_eq = cols == rows
    dist = jnp.where(self_eq, dist + 1e10, dist)
    mask = self_eq
    for _ in range(K):
        mval = jnp.min(dist, axis=1, keepdims=True)
        cand = jnp.where(dist == mval, cols, n)
        idx = jnp.min(cand, axis=1, keepdims=True)
        eq = cols == idx
        mask = mask | eq
        dist = jnp.where(eq, jnp.float32(jnp.inf), dist)
    out_ref[0] = mask.astype(jnp.int8)


def _flash_step(xw_ref, as_ref, adt_ref, msk_ref, m_scr, l_scr, acc_scr, *, n):
    t = pl.program_id(1)

    @pl.when(t == 0)
    def _init():
        m_scr[...] = jnp.full((8, n), NEG, jnp.float32)
        l_scr[...] = jnp.zeros((8, n), jnp.float32)
        acc_scr[...] = jnp.zeros((HID, n), jnp.float32)

    mask = msk_ref[0].astype(jnp.int32) > 0
    xw = xw_ref[...]
    ad_all = adt_ref[0]
    for h in range(H):
        a = as_ref[:, h:h + 1] + ad_all[h:h + 1, :]
        a = jnp.maximum(a, 0.2 * a)
        a = jnp.where(mask, a, NEG)
        m_old = m_scr[h:h + 1, :]
        m_new = jnp.maximum(m_old, jnp.max(a, axis=0, keepdims=True))
        corr = jnp.exp(m_old - m_new)
        e = jnp.exp(a - m_new)
        l_scr[h:h + 1, :] = l_scr[h:h + 1, :] * corr + jnp.sum(
            e, axis=0, keepdims=True)
        m_scr[h:h + 1, :] = m_new
        sl = slice(C * h, C * (h + 1))
        acc_scr[sl, :] = acc_scr[sl, :] * corr + _tdot(xw[:, sl], e)


def _ln_relu(y, g, b):
    mu = jnp.mean(y, axis=0, keepdims=True)
    var = jnp.mean((y - mu) * (y - mu), axis=0, keepdims=True)
    return jnp.maximum((y - mu) / jnp.sqrt(var + 1e-5) * g + b, 0.0)


def _gat_out(acc_scr, l_scr, bias_ref, g_ref, bt_ref, *, n):
    l_full = jnp.concatenate(
        [jnp.broadcast_to(l_scr[h:h + 1, :], (C, n)) for h in range(H)],
        axis=0)
    y = acc_scr[...] / (l_full + 1e-16) + bias_ref[...]
    return _ln_relu(y, g_ref[...], bt_ref[...])


def _gat0_kernel(xw_ref, as_ref, adt_ref, knn_ref, bias_ref, g_ref, bt_ref,
                 w1_ref, as1_ref, ad1_ref, xw1t_ref, asad1_ref,
                 m_scr, l_scr, acc_scr, *, n, nt):
    _flash_step(xw_ref, as_ref, adt_ref, knn_ref, m_scr, l_scr, acc_scr, n=n)

    @pl.when(pl.program_id(1) == nt - 1)
    def _fin():
        x1 = _gat_out(acc_scr, l_scr, bias_ref, g_ref, bt_ref, n=n)
        xw1t = _tdot(w1_ref[...], x1)
        as1t = _tdot(as1_ref[...], xw1t)
        ad1t = _tdot(ad1_ref[...], xw1t)
        xw1t_ref[0] = xw1t
        asad1_ref[0] = jnp.concatenate([as1t, ad1t], axis=0)


def _gat1_kernel(xw_ref, as_ref, adt_ref, knn_ref, bias_ref, g_ref, bt_ref,
                 aw1_ref, ab1_ref, aw2_ref, ab2_ref,
                 cw1_ref, cb1_ref, cw2_ref, cb2_ref, head_ref,
                 m_scr, l_scr, acc_scr, *, n, nt):
    _flash_step(xw_ref, as_ref, adt_ref, knn_ref, m_scr, l_scr, acc_scr, n=n)

    @pl.when(pl.program_id(1) == nt - 1)
    def _fin():
        x2 = _gat_out(acc_scr, l_scr, bias_ref, g_ref, bt_ref, n=n)
        h_a = jnp.maximum(_tdot(aw1_ref[...], x2) + ab1_ref[...], 0.0)
        act = jnp.tanh(_tdot(aw2_ref[...], h_a) + ab2_ref[...])
        h_c = jnp.maximum(_tdot(cw1_ref[...], x2) + cb1_ref[...], 0.0)
        val = _tdot(cw2_ref[...], h_c) + cb2_ref[...]
        riota = lax.broadcasted_iota(jnp.int32, (8, n), 0)
        head_ref[0] = jnp.where(riota == 3, val, act)


def _mix(a):
    return (jnp.eye(H, dtype=a.dtype)[:, None, :] * a[:, :, None]).reshape(
        H * C, H)


@jax.jit
def kernel(obs, enc_w, enc_b, enc_g, enc_bt, tem_w, tem_b,
           g0_w, g0_as, g0_ad, g0_b, g0_g, g0_bt,
           g1_w, g1_as, g1_ad, g1_b, g1_g, g1_bt,
           act_w1, act_b1, act_w2, act_b2, cr_w1, cr_b1, cr_w2, cr_b2):
    B, N, D = obs.shape
    Nt = B * N
    NT = N // TILE
    NB = Nt // TILE
    f32 = jnp.float32

    flat = obs.reshape(Nt, D)
    pos_t = jnp.transpose(obs[:, :, :3], (0, 2, 1))
    p2 = jnp.sum(pos_t * pos_t, axis=1)
    p2r = p2[:, None, :]
    p2c = p2[:, :, None]

    row2 = lambda v: v.reshape(1, -1)
    col2 = lambda v: v.reshape(-1, 1)
    full = lambda a: pl.BlockSpec(a.shape, lambda b, t: (0,) * a.ndim)

    As0, Ad0 = _mix(g0_as), _mix(g0_ad)
    xw0, asad0 = pl.pallas_call(
        _enc_kernel,
        grid=(NB,),
        in_specs=[
            pl.BlockSpec((TILE, D), lambda i: (i, 0)),
            pl.BlockSpec((9, HID), lambda i: (0, 0)),
            pl.BlockSpec((1, HID), lambda i: (0, 0)),
            pl.BlockSpec((1, HID), lambda i: (0, 0)),
            pl.BlockSpec((1, HID), lambda i: (0, 0)),
            pl.BlockSpec((1, HID), lambda i: (0, 0)),
            pl.BlockSpec((1, HID), lambda i: (0, 0)),
            pl.BlockSpec((HID, HID), lambda i: (0, 0)),
            pl.BlockSpec((HID, H), lambda i: (0, 0)),
            pl.BlockSpec((HID, H), lambda i: (0, 0)),
        ],
        out_specs=[
            pl.BlockSpec((TILE, HID), lambda i: (i, 0)),
            pl.BlockSpec((TILE, 2 * H), lambda i: (i, 0)),
        ],
        out_shape=[
            jax.ShapeDtypeStruct((Nt, HID), f32),
            jax.ShapeDtypeStruct((Nt, 2 * H), f32),
        ],
    )(flat, enc_w, row2(enc_b), row2(enc_g), row2(enc_bt),
      row2(tem_w), row2(tem_b), g0_w, As0, Ad0)

    adj = pl.pallas_call(
        functools.partial(_knn_kernel, n=N),
        grid=(B, NT),
        in_specs=[
            pl.BlockSpec((1, 3, TILE), lambda b, t: (b, 0, t)),
            pl.BlockSpec((1, 3, N), lambda b, t: (b, 0, 0)),
            pl.BlockSpec((1, 1, N), lambda b, t: (b, 0, 0)),
            pl.BlockSpec((1, TILE, 1), lambda b, t: (b, t, 0)),
        ],
        out_specs=pl.BlockSpec((1, TILE, N), lambda b, t: (b, t, 0)),
        out_shape=jax.ShapeDtypeStruct((B, N, N), jnp.int8),
    )(pos_t, pos_t, p2r, p2c)

    scratch = [
        pltpu.VMEM((8, N), f32),
        pltpu.VMEM((8, N), f32),
        pltpu.VMEM((HID, N), f32),
    ]
    flash_specs = [
        pl.BlockSpec((TILE, HID), lambda b, t: (b * NT + t, 0)),
        pl.BlockSpec((TILE, H), lambda b, t: (b * NT + t, 0)),
        pl.BlockSpec((1, H, N), lambda b, t: (b, 0, 0)),
        pl.BlockSpec((1, TILE, N), lambda b, t: (b, t, 0)),
    ]

    As1, Ad1 = _mix(g1_as), _mix(g1_ad)
    as0 = asad0[:, :H]
    ad0t = jnp.transpose(asad0[:, H:].reshape(B, N, H), (0, 2, 1))
    xw1t, asad1t = pl.pallas_call(
        functools.partial(_gat0_kernel, n=N, nt=NT),
        grid=(B, NT),
        in_specs=flash_specs + [
            full(col2(g0_b)), full(col2(g0_g)), full(col2(g0_bt)),
            full(g1_w), full(As1), full(Ad1),
        ],
        out_specs=[
            pl.BlockSpec((1, HID, N), lambda b, t: (b, 0, 0)),
            pl.BlockSpec((1, 2 * H, N), lambda b, t: (b, 0, 0)),
        ],
        out_shape=[
            jax.ShapeDtypeStruct((B, HID, N), f32),
            jax.ShapeDtypeStruct((B, 2 * H, N), f32),
        ],
        scratch_shapes=scratch,
    )(xw0, as0, ad0t, adj, col2(g0_b), col2(g0_g), col2(g0_bt),
      g1_w, As1, Ad1)

    xw1 = jnp.transpose(xw1t, (0, 2, 1)).reshape(Nt, HID)
    as1 = jnp.transpose(asad1t[:, :H, :], (0, 2, 1)).reshape(Nt, H)
    ad1t = asad1t[:, H:, :]
    aw2 = jnp.pad(act_w2, ((0, 0), (0, 8 - act_w2.shape[1])))
    ab2 = col2(jnp.pad(act_b2, (0, 8 - act_b2.shape[0])))
    cw2 = jnp.pad(cr_w2, ((0, 0), (3, 4)))
    cb2 = col2(jnp.pad(cr_b2, (3, 4)))
    head = pl.pallas_call(
        functools.partial(_gat1_kernel, n=N, nt=NT),
        grid=(B, NT),
        in_specs=flash_specs + [
            full(col2(g1_b)), full(col2(g1_g)), full(col2(g1_bt)),
            full(act_w1), full(col2(act_b1)), full(aw2), full(ab2),
            full(cr_w1), full(col2(cr_b1)), full(cw2), full(cb2),
        ],
        out_specs=pl.BlockSpec((1, 8, N), lambda b, t: (b, 0, 0)),
        out_shape=jax.ShapeDtypeStruct((B, 8, N), f32),
        scratch_shapes=scratch,
    )(xw1, as1, ad1t, adj, col2(g1_b), col2(g1_g), col2(g1_bt),
      act_w1, col2(act_b1), aw2, ab2, cr_w1, col2(cr_b1), cw2, cb2)

    action = jnp.transpose(head[:, :3, :], (0, 2, 1))
    value = jnp.transpose(head[:, 3:4, :], (0, 2, 1))
    return (action, value)


# speedup vs baseline: 50.7588x; 1.0693x over previous
import functools

import jax
import jax.numpy as jnp
from jax import lax
from jax.experimental import pallas as pl
from jax.experimental.pallas import tpu as pltpu

H = 4
C = 32
HID = 128
K = 10
TILE = 256
NEG = -1e30

_DN = (((0,), (0,)), ((), ()))


def _tdot(a, b, precision=None):
    return lax.dot_general(a, b, _DN, precision=precision,
                           preferred_element_type=jnp.float32)


def _enc_kernel(obs_ref, encw_ref, encb_ref, encg_ref, encbt_ref,
                temw_ref, temb_ref, w0_ref, as0_ref, ad0_ref,
                xw_ref, asad_ref):
    blk = obs_ref[...]
    phys = blk[:, :9]
    lat = blk[:, 9:10]
    e = jnp.maximum(
        jnp.dot(phys, encw_ref[...], preferred_element_type=jnp.float32)
        + encb_ref[...], 0.0)
    mu = jnp.mean(e, axis=1, keepdims=True)
    var = jnp.mean((e - mu) * (e - mu), axis=1, keepdims=True)
    x = (e - mu) / jnp.sqrt(var + 1e-5) * encg_ref[...] + encbt_ref[...]
    t = lat * temw_ref[...] + temb_ref[...]
    x = x + t * (1.0 / (1.0 + jnp.exp(-t)))
    xw = jnp.dot(x, w0_ref[...], preferred_element_type=jnp.float32)
    a_s = jnp.dot(xw, as0_ref[...], preferred_element_type=jnp.float32)
    a_d = jnp.dot(xw, ad0_ref[...], preferred_element_type=jnp.float32)
    xw_ref[...] = xw
    asad_ref[...] = jnp.concatenate([a_s, a_d], axis=1)


def _knn_kernel(post_ref, posa_ref, p2r_ref, p2c_ref, out_ref, *, n):
    pos_t = post_ref[0]
    pos_a = posa_ref[0]
    cross = _tdot(pos_t, pos_a)
    d2 = p2c_ref[0] + p2r_ref[0] - 2.0 * cross
    dist = jnp.sqrt(jnp.maximum(d2, 0.0))
    t = pl.program_id(1)
    rows = t * TILE + lax.broadcasted_iota(jnp.int32, (TILE, 1), 0)
    cols = lax.broadcasted_iota(jnp.int32, (TILE, n), 1)
    self_eq = cols == rows
    dist = jnp.where(self_eq, dist + 1e10, dist)
    mask = self_eq
    for _ in range(K):
        mval = jnp.min(dist, axis=1, keepdims=True)
        cand = jnp.where(dist == mval, cols, n)
        idx = jnp.min(cand, axis=1, keepdims=True)
        eq = cols == idx
        mask = mask | eq
        dist = jnp.where(eq, jnp.float32(jnp.inf), dist)
    out_ref[0] = mask.astype(jnp.int8)


def _flash_step(xw_ref, as_ref, adb_ref, msk_ref, l_scr, acc_scr, *, n):
    t = pl.program_id(1)

    @pl.when(t == 0)
    def _init():
        l_scr[...] = jnp.zeros((8, n), jnp.float32)
        acc_scr[...] = jnp.zeros((HID, n), jnp.float32)

    mask = msk_ref[0].astype(jnp.int32) > 0
    xw = xw_ref[...]
    adb = adb_ref[0]
    for h in range(H):
        a = as_ref[:, h:h + 1] + adb[h:h + 1, :]
        a = jnp.maximum(a, 0.2 * a)
        a = jnp.where(mask, a, NEG)
        e = jnp.exp(a - adb[H + h:H + h + 1, :])
        l_scr[h:h + 1, :] = l_scr[h:h + 1, :] + jnp.sum(
            e, axis=0, keepdims=True)
        sl = slice(C * h, C * (h + 1))
        acc_scr[sl, :] = acc_scr[sl, :] + _tdot(xw[:, sl], e)


def _ln_relu(y, g, b):
    mu = jnp.mean(y, axis=0, keepdims=True)
    var = jnp.mean((y - mu) * (y - mu), axis=0, keepdims=True)
    return jnp.maximum((y - mu) / jnp.sqrt(var + 1e-5) * g + b, 0.0)


def _gat_out(acc_scr, l_scr, bias_ref, g_ref, bt_ref, *, n):
    l_full = jnp.concatenate(
        [jnp.broadcast_to(l_scr[h:h + 1, :], (C, n)) for h in range(H)],
        axis=0)
    y = acc_scr[...] / (l_full + 1e-16) + bias_ref[...]
    return _ln_relu(y, g_ref[...], bt_ref[...])


def _gat0_kernel(xw_ref, as_ref, adb_ref, msk_ref, bias_ref, g_ref, bt_ref,
                 w1_ref, as1_ref, ad1_ref, xw1t_ref, asad1_ref,
                 l_scr, acc_scr, *, n, nt):
    _flash_step(xw_ref, as_ref, adb_ref, msk_ref, l_scr, acc_scr, n=n)

    @pl.when(pl.program_id(1) == nt - 1)
    def _fin():
        x1 = _gat_out(acc_scr, l_scr, bias_ref, g_ref, bt_ref, n=n)
        xw1t = _tdot(w1_ref[...], x1)
        as1t = _tdot(as1_ref[...], xw1t)
        ad1t = _tdot(ad1_ref[...], xw1t)
        asmax = jnp.max(as1t, axis=1, keepdims=True)
        b1t = asmax + ad1t
        b1t = jnp.maximum(b1t, 0.2 * b1t)
        xw1t_ref[0] = xw1t
        asad1_ref[0] = jnp.concatenate(
            [as1t, ad1t, b1t, jnp.zeros((4, n), jnp.float32)], axis=0)


def _gat1_kernel(xw_ref, as_ref, adb_ref, msk_ref, bias_ref, g_ref, bt_ref,
                 aw1_ref, ab1_ref, aw2_ref, ab2_ref,
                 cw1_ref, cb1_ref, cw2_ref, cb2_ref, head_ref,
                 l_scr, acc_scr, *, n, nt):
    _flash_step(xw_ref, as_ref, adb_ref, msk_ref, l_scr, acc_scr, n=n)

    @pl.when(pl.program_id(1) == nt - 1)
    def _fin():
        x2 = _gat_out(acc_scr, l_scr, bias_ref, g_ref, bt_ref, n=n)
        h_a = jnp.maximum(_tdot(aw1_ref[...], x2) + ab1_ref[...], 0.0)
        act = jnp.tanh(_tdot(aw2_ref[...], h_a) + ab2_ref[...])
        h_c = jnp.maximum(_tdot(cw1_ref[...], x2) + cb1_ref[...], 0.0)
        val = _tdot(cw2_ref[...], h_c) + cb2_ref[...]
        riota = lax.broadcasted_iota(jnp.int32, (8, n), 0)
        head_ref[0] = jnp.where(riota == 3, val, act)


def _mix(a):
    return (jnp.eye(H, dtype=a.dtype)[:, None, :] * a[:, :, None]).reshape(
        H * C, H)


@jax.jit
def kernel(obs, enc_w, enc_b, enc_g, enc_bt, tem_w, tem_b,
           g0_w, g0_as, g0_ad, g0_b, g0_g, g0_bt,
           g1_w, g1_as, g1_ad, g1_b, g1_g, g1_bt,
           act_w1, act_b1, act_w2, act_b2, cr_w1, cr_b1, cr_w2, cr_b2):
    B, N, D = obs.shape
    Nt = B * N
    NT = N // TILE
    NB = Nt // TILE
    f32 = jnp.float32

    flat = obs.reshape(Nt, D)
    pos_t = jnp.transpose(obs[:, :, :3], (0, 2, 1))
    p2 = jnp.sum(pos_t * pos_t, axis=1)
    p2r = p2[:, None, :]
    p2c = p2[:, :, None]

    row2 = lambda v: v.reshape(1, -1)
    col2 = lambda v: v.reshape(-1, 1)
    full = lambda a: pl.BlockSpec(a.shape, lambda b, t: (0,) * a.ndim)

    As0, Ad0 = _mix(g0_as), _mix(g0_ad)
    xw0, asad0 = pl.pallas_call(
        _enc_kernel,
        grid=(NB,),
        in_specs=[
            pl.BlockSpec((TILE, D), lambda i: (i, 0)),
            pl.BlockSpec((9, HID), lambda i: (0, 0)),
            pl.BlockSpec((1, HID), lambda i: (0, 0)),
            pl.BlockSpec((1, HID), lambda i: (0, 0)),
            pl.BlockSpec((1, HID), lambda i: (0, 0)),
            pl.BlockSpec((1, HID), lambda i: (0, 0)),
            pl.BlockSpec((1, HID), lambda i: (0, 0)),
            pl.BlockSpec((HID, HID), lambda i: (0, 0)),
            pl.BlockSpec((HID, H), lambda i: (0, 0)),
            pl.BlockSpec((HID, H), lambda i: (0, 0)),
        ],
        out_specs=[
            pl.BlockSpec((TILE, HID), lambda i: (i, 0)),
            pl.BlockSpec((TILE, 2 * H), lambda i: (i, 0)),
        ],
        out_shape=[
            jax.ShapeDtypeStruct((Nt, HID), f32),
            jax.ShapeDtypeStruct((Nt, 2 * H), f32),
        ],
    )(flat, enc_w, row2(enc_b), row2(enc_g), row2(enc_bt),
      row2(tem_w), row2(tem_b), g0_w, As0, Ad0)

    adj = pl.pallas_call(
        functools.partial(_knn_kernel, n=N),
        grid=(B, NT),
        in_specs=[
            pl.BlockSpec((1, 3, TILE), lambda b, t: (b, 0, t)),
            pl.BlockSpec((1, 3, N), lambda b, t: (b, 0, 0)),
            pl.BlockSpec((1, 1, N), lambda b, t: (b, 0, 0)),
            pl.BlockSpec((1, TILE, 1), lambda b, t: (b, t, 0)),
        ],
        out_specs=pl.BlockSpec((1, TILE, N), lambda b, t: (b, t, 0)),
        out_shape=jax.ShapeDtypeStruct((B, N, N), jnp.int8),
    )(pos_t, pos_t, p2r, p2c)

    scratch = [
        pltpu.VMEM((8, N), f32),
        pltpu.VMEM((HID, N), f32),
    ]
    flash_specs = [
        pl.BlockSpec((TILE, HID), lambda b, t: (b * NT + t, 0)),
        pl.BlockSpec((TILE, H), lambda b, t: (b * NT + t, 0)),
        pl.BlockSpec((1, 2 * H, N), lambda b, t: (b, 0, 0)),
        pl.BlockSpec((1, TILE, N), lambda b, t: (b, t, 0)),
    ]

    As1, Ad1 = _mix(g1_as), _mix(g1_ad)
    as0 = asad0[:, :H]
    ad0t = jnp.transpose(asad0[:, H:].reshape(B, N, H), (0, 2, 1))
    b0t = jnp.max(as0.reshape(B, N, H), axis=1)[:, :, None] + ad0t
    adb0t = jnp.concatenate([ad0t, jnp.maximum(b0t, 0.2 * b0t)], axis=1)
    xw1t, asad1t = pl.pallas_call(
        functools.partial(_gat0_kernel, n=N, nt=NT),
        grid=(B, NT),
        in_specs=flash_specs + [
            full(col2(g0_b)), full(col2(g0_g)), full(col2(g0_bt)),
            full(g1_w), full(As1), full(Ad1),
        ],
        out_specs=[
            pl.BlockSpec((1, HID, N), lambda b, t: (b, 0, 0)),
            pl.BlockSpec((1, 4 * H, N), lambda b, t: (b, 0, 0)),
        ],
        out_shape=[
            jax.ShapeDtypeStruct((B, HID, N), f32),
            jax.ShapeDtypeStruct((B, 4 * H, N), f32),
        ],
        scratch_shapes=scratch,
    )(xw0, as0, adb0t, adj, col2(g0_b), col2(g0_g), col2(g0_bt),
      g1_w, As1, Ad1)

    xw1 = jnp.transpose(xw1t, (0, 2, 1)).reshape(Nt, HID)
    as1 = jnp.transpose(asad1t[:, :H, :], (0, 2, 1)).reshape(Nt, H)
    adb1t = asad1t[:, H:3 * H, :]
    aw2 = jnp.pad(act_w2, ((0, 0), (0, 8 - act_w2.shape[1])))
    ab2 = col2(jnp.pad(act_b2, (0, 8 - act_b2.shape[0])))
    cw2 = jnp.pad(cr_w2, ((0, 0), (3, 4)))
    cb2 = col2(jnp.pad(cr_b2, (3, 4)))
    head = pl.pallas_call(
        functools.partial(_gat1_kernel, n=N, nt=NT),
        grid=(B, NT),
        in_specs=flash_specs + [
            full(col2(g1_b)), full(col2(g1_g)), full(col2(g1_bt)),
            full(act_w1), full(col2(act_b1)), full(aw2), full(ab2),
            full(cr_w1), full(col2(cr_b1)), full(cw2), full(cb2),
        ],
        out_specs=pl.BlockSpec((1, 8, N), lambda b, t: (b, 0, 0)),
        out_shape=jax.ShapeDtypeStruct((B, 8, N), f32),
        scratch_shapes=scratch,
    )(xw1, as1, adb1t, adj, col2(g1_b), col2(g1_g), col2(g1_bt),
      act_w1, col2(act_b1), aw2, ab2, cr_w1, col2(cr_b1), cw2, cb2)

    action = jnp.transpose(head[:, :3, :], (0, 2, 1))
    value = jnp.transpose(head[:, 3:4, :], (0, 2, 1))
    return (action, value)


# speedup vs baseline: 57.9822x; 1.1423x over previous
import functools

import jax
import jax.numpy as jnp
from jax import lax
from jax.experimental import pallas as pl
from jax.experimental.pallas import tpu as pltpu

H = 4
C = 32
HID = 128
K = 10
TILE = 512
NEG = -1e30

_DN = (((0,), (0,)), ((), ()))


def _tdot(a, b, precision=None):
    return lax.dot_general(a, b, _DN, precision=precision,
                           preferred_element_type=jnp.float32)


def _enc_kernel(obs_ref, encw_ref, encb_ref, encg_ref, encbt_ref,
                temw_ref, temb_ref, w0_ref, as0_ref, ad0_ref,
                xw_ref, asad_ref):
    blk = obs_ref[...]
    phys = blk[:, :9]
    lat = blk[:, 9:10]
    e = jnp.maximum(
        jnp.dot(phys, encw_ref[...], preferred_element_type=jnp.float32)
        + encb_ref[...], 0.0)
    mu = jnp.mean(e, axis=1, keepdims=True)
    var = jnp.mean((e - mu) * (e - mu), axis=1, keepdims=True)
    x = (e - mu) / jnp.sqrt(var + 1e-5) * encg_ref[...] + encbt_ref[...]
    t = lat * temw_ref[...] + temb_ref[...]
    x = x + t * (1.0 / (1.0 + jnp.exp(-t)))
    xw = jnp.dot(x, w0_ref[...], preferred_element_type=jnp.float32)
    a_s = jnp.dot(xw, as0_ref[...], preferred_element_type=jnp.float32)
    a_d = jnp.dot(xw, ad0_ref[...], preferred_element_type=jnp.float32)
    xw_ref[...] = xw
    asad_ref[...] = jnp.concatenate([a_s, a_d], axis=1)


def _knn_kernel(post_ref, posa_ref, p2r_ref, p2c_ref, out_ref, *, n):
    pos_t = post_ref[0]
    pos_a = posa_ref[0]
    cross = _tdot(pos_t, pos_a)
    d2 = p2c_ref[0] + p2r_ref[0] - 2.0 * cross
    dist = jnp.sqrt(jnp.maximum(d2, 0.0))
    t = pl.program_id(1)
    rows = t * TILE + lax.broadcasted_iota(jnp.int32, (TILE, 1), 0)
    cols = lax.broadcasted_iota(jnp.int32, (TILE, n), 1)
    self_eq = cols == rows
    dist = jnp.where(self_eq, dist + 1e10, dist)
    mask = self_eq
    for _ in range(K):
        idx = jnp.argmin(dist, axis=1)[:, None]
        eq = cols == idx
        mask = mask | eq
        dist = jnp.where(eq, jnp.float32(jnp.inf), dist)
    out_ref[0] = mask.astype(jnp.int8)


def _flash_step(xw_ref, as_ref, adb_ref, msk_ref, l_scr, acc_scr, *, n):
    t = pl.program_id(1)

    @pl.when(t == 0)
    def _init():
        l_scr[...] = jnp.zeros((8, n), jnp.float32)
        acc_scr[...] = jnp.zeros((HID, n), jnp.float32)

    mask = msk_ref[0].astype(jnp.int32) > 0
    xw = xw_ref[...]
    adb = adb_ref[0]
    for h in range(H):
        a = as_ref[:, h:h + 1] + adb[h:h + 1, :]
        a = jnp.maximum(a, 0.2 * a)
        a = jnp.where(mask, a, NEG)
        e = jnp.exp(a - adb[H + h:H + h + 1, :])
        l_scr[h:h + 1, :] = l_scr[h:h + 1, :] + jnp.sum(
            e, axis=0, keepdims=True)
        sl = slice(C * h, C * (h + 1))
        acc_scr[sl, :] = acc_scr[sl, :] + _tdot(xw[:, sl], e)


def _ln_relu(y, g, b):
    mu = jnp.mean(y, axis=0, keepdims=True)
    var = jnp.mean((y - mu) * (y - mu), axis=0, keepdims=True)
    return jnp.maximum((y - mu) / jnp.sqrt(var + 1e-5) * g + b, 0.0)


def _gat_out(acc_scr, l_scr, bias_ref, g_ref, bt_ref, *, n):
    l_full = jnp.concatenate(
        [jnp.broadcast_to(l_scr[h:h + 1, :], (C, n)) for h in range(H)],
        axis=0)
    y = acc_scr[...] / (l_full + 1e-16) + bias_ref[...]
    return _ln_relu(y, g_ref[...], bt_ref[...])


def _gat0_kernel(xw_ref, as_ref, adb_ref, msk_ref, bias_ref, g_ref, bt_ref,
                 w1_ref, as1_ref, ad1_ref, xw1t_ref, asad1_ref,
                 l_scr, acc_scr, *, n, nt):
    _flash_step(xw_ref, as_ref, adb_ref, msk_ref, l_scr, acc_scr, n=n)

    @pl.when(pl.program_id(1) == nt - 1)
    def _fin():
        x1 = _gat_out(acc_scr, l_scr, bias_ref, g_ref, bt_ref, n=n)
        xw1t = _tdot(w1_ref[...], x1)
        as1t = _tdot(as1_ref[...], xw1t)
        ad1t = _tdot(ad1_ref[...], xw1t)
        asmax = jnp.max(as1t, axis=1, keepdims=True)
        b1t = asmax + ad1t
        b1t = jnp.maximum(b1t, 0.2 * b1t)
        xw1t_ref[0] = xw1t
        asad1_ref[0] = jnp.concatenate(
            [as1t, ad1t, b1t, jnp.zeros((4, n), jnp.float32)], axis=0)


def _gat1_kernel(xw_ref, as_ref, adb_ref, msk_ref, bias_ref, g_ref, bt_ref,
                 aw1_ref, ab1_ref, aw2_ref, ab2_ref,
                 cw1_ref, cb1_ref, cw2_ref, cb2_ref, head_ref,
                 l_scr, acc_scr, *, n, nt):
    _flash_step(xw_ref, as_ref, adb_ref, msk_ref, l_scr, acc_scr, n=n)

    @pl.when(pl.program_id(1) == nt - 1)
    def _fin():
        x2 = _gat_out(acc_scr, l_scr, bias_ref, g_ref, bt_ref, n=n)
        h_a = jnp.maximum(_tdot(aw1_ref[...], x2) + ab1_ref[...], 0.0)
        act = jnp.tanh(_tdot(aw2_ref[...], h_a) + ab2_ref[...])
        h_c = jnp.maximum(_tdot(cw1_ref[...], x2) + cb1_ref[...], 0.0)
        val = _tdot(cw2_ref[...], h_c) + cb2_ref[...]
        riota = lax.broadcasted_iota(jnp.int32, (8, n), 0)
        head_ref[0] = jnp.where(riota == 3, val, act)


def _mix(a):
    return (jnp.eye(H, dtype=a.dtype)[:, None, :] * a[:, :, None]).reshape(
        H * C, H)


@jax.jit
def kernel(obs, enc_w, enc_b, enc_g, enc_bt, tem_w, tem_b,
           g0_w, g0_as, g0_ad, g0_b, g0_g, g0_bt,
           g1_w, g1_as, g1_ad, g1_b, g1_g, g1_bt,
           act_w1, act_b1, act_w2, act_b2, cr_w1, cr_b1, cr_w2, cr_b2):
    B, N, D = obs.shape
    Nt = B * N
    NT = N // TILE
    NB = Nt // TILE
    f32 = jnp.float32

    flat = obs.reshape(Nt, D)
    pos_t = jnp.transpose(obs[:, :, :3], (0, 2, 1))
    p2 = jnp.sum(pos_t * pos_t, axis=1)
    p2r = p2[:, None, :]
    p2c = p2[:, :, None]

    row2 = lambda v: v.reshape(1, -1)
    col2 = lambda v: v.reshape(-1, 1)
    full = lambda a: pl.BlockSpec(a.shape, lambda b, t: (0,) * a.ndim)

    As0, Ad0 = _mix(g0_as), _mix(g0_ad)
    xw0, asad0 = pl.pallas_call(
        _enc_kernel,
        grid=(NB,),
        in_specs=[
            pl.BlockSpec((TILE, D), lambda i: (i, 0)),
            pl.BlockSpec((9, HID), lambda i: (0, 0)),
            pl.BlockSpec((1, HID), lambda i: (0, 0)),
            pl.BlockSpec((1, HID), lambda i: (0, 0)),
            pl.BlockSpec((1, HID), lambda i: (0, 0)),
            pl.BlockSpec((1, HID), lambda i: (0, 0)),
            pl.BlockSpec((1, HID), lambda i: (0, 0)),
            pl.BlockSpec((HID, HID), lambda i: (0, 0)),
            pl.BlockSpec((HID, H), lambda i: (0, 0)),
            pl.BlockSpec((HID, H), lambda i: (0, 0)),
        ],
        out_specs=[
            pl.BlockSpec((TILE, HID), lambda i: (i, 0)),
            pl.BlockSpec((TILE, 2 * H), lambda i: (i, 0)),
        ],
        out_shape=[
            jax.ShapeDtypeStruct((Nt, HID), f32),
            jax.ShapeDtypeStruct((Nt, 2 * H), f32),
        ],
    )(flat, enc_w, row2(enc_b), row2(enc_g), row2(enc_bt),
      row2(tem_w), row2(tem_b), g0_w, As0, Ad0)

    adj = pl.pallas_call(
        functools.partial(_knn_kernel, n=N),
        grid=(B, NT),
        in_specs=[
            pl.BlockSpec((1, 3, TILE), lambda b, t: (b, 0, t)),
            pl.BlockSpec((1, 3, N), lambda b, t: (b, 0, 0)),
            pl.BlockSpec((1, 1, N), lambda b, t: (b, 0, 0)),
            pl.BlockSpec((1, TILE, 1), lambda b, t: (b, t, 0)),
        ],
        out_specs=pl.BlockSpec((1, TILE, N), lambda b, t: (b, t, 0)),
        out_shape=jax.ShapeDtypeStruct((B, N, N), jnp.int8),
    )(pos_t, pos_t, p2r, p2c)

    scratch = [
        pltpu.VMEM((8, N), f32),
        pltpu.VMEM((HID, N), f32),
    ]
    flash_specs = [
        pl.BlockSpec((TILE, HID), lambda b, t: (b * NT + t, 0)),
        pl.BlockSpec((TILE, H), lambda b, t: (b * NT + t, 0)),
        pl.BlockSpec((1, 2 * H, N), lambda b, t: (b, 0, 0)),
        pl.BlockSpec((1, TILE, N), lambda b, t: (b, t, 0)),
    ]

    As1, Ad1 = _mix(g1_as), _mix(g1_ad)
    as0 = asad0[:, :H]
    ad0t = jnp.transpose(asad0[:, H:].reshape(B, N, H), (0, 2, 1))
    b0t = jnp.max(as0.reshape(B, N, H), axis=1)[:, :, None] + ad0t
    adb0t = jnp.concatenate([ad0t, jnp.maximum(b0t, 0.2 * b0t)], axis=1)
    xw1t, asad1t = pl.pallas_call(
        functools.partial(_gat0_kernel, n=N, nt=NT),
        grid=(B, NT),
        in_specs=flash_specs + [
            full(col2(g0_b)), full(col2(g0_g)), full(col2(g0_bt)),
            full(g1_w), full(As1), full(Ad1),
        ],
        out_specs=[
            pl.BlockSpec((1, HID, N), lambda b, t: (b, 0, 0)),
            pl.BlockSpec((1, 4 * H, N), lambda b, t: (b, 0, 0)),
        ],
        out_shape=[
            jax.ShapeDtypeStruct((B, HID, N), f32),
            jax.ShapeDtypeStruct((B, 4 * H, N), f32),
        ],
        scratch_shapes=scratch,
    )(xw0, as0, adb0t, adj, col2(g0_b), col2(g0_g), col2(g0_bt),
      g1_w, As1, Ad1)

    xw1 = jnp.transpose(xw1t, (0, 2, 1)).reshape(Nt, HID)
    as1 = jnp.transpose(asad1t[:, :H, :], (0, 2, 1)).reshape(Nt, H)
    adb1t = asad1t[:, H:3 * H, :]
    aw2 = jnp.pad(act_w2, ((0, 0), (0, 8 - act_w2.shape[1])))
    ab2 = col2(jnp.pad(act_b2, (0, 8 - act_b2.shape[0])))
    cw2 = jnp.pad(cr_w2, ((0, 0), (3, 4)))
    cb2 = col2(jnp.pad(cr_b2, (3, 4)))
    head = pl.pallas_call(
        functools.partial(_gat1_kernel, n=N, nt=NT),
        grid=(B, NT),
        in_specs=flash_specs + [
            full(col2(g1_b)), full(col2(g1_g)), full(col2(g1_bt)),
            full(act_w1), full(col2(act_b1)), full(aw2), full(ab2),
            full(cr_w1), full(col2(cr_b1)), full(cw2), full(cb2),
        ],
        out_specs=pl.BlockSpec((1, 8, N), lambda b, t: (b, 0, 0)),
        out_shape=jax.ShapeDtypeStruct((B, 8, N), f32),
        scratch_shapes=scratch,
    )(xw1, as1, adb1t, adj, col2(g1_b), col2(g1_g), col2(g1_bt),
      act_w1, col2(act_b1), aw2, ab2, cr_w1, col2(cr_b1), cw2, cb2)

    action = jnp.transpose(head[:, :3, :], (0, 2, 1))
    value = jnp.transpose(head[:, 3:4, :], (0, 2, 1))
    return (action, value)


# speedup vs baseline: 66.8731x; 1.1533x over previous
import functools

import jax
import jax.numpy as jnp
from jax import lax
from jax.experimental import pallas as pl
from jax.experimental.pallas import tpu as pltpu

H = 4
C = 32
HID = 128
K = 10
TILE = 512
NEG = -1e30

_DN = (((0,), (0,)), ((), ()))


def _tdot(a, b, precision=None):
    return lax.dot_general(a, b, _DN, precision=precision,
                           preferred_element_type=jnp.float32)


def _enc_kernel(obs_ref, encw_ref, encb_ref, encg_ref, encbt_ref,
                temw_ref, temb_ref, w0_ref, as0_ref, ad0_ref,
                xw_ref, asad_ref):
    blk = obs_ref[...]
    phys = blk[:, :9]
    lat = blk[:, 9:10]
    e = jnp.maximum(
        jnp.dot(phys, encw_ref[...], preferred_element_type=jnp.float32)
        + encb_ref[...], 0.0)
    mu = jnp.mean(e, axis=1, keepdims=True)
    var = jnp.mean((e - mu) * (e - mu), axis=1, keepdims=True)
    x = (e - mu) / jnp.sqrt(var + 1e-5) * encg_ref[...] + encbt_ref[...]
    t = lat * temw_ref[...] + temb_ref[...]
    x = x + t * (1.0 / (1.0 + jnp.exp(-t)))
    xw = jnp.dot(x, w0_ref[...], preferred_element_type=jnp.float32)
    a_s = jnp.dot(xw, as0_ref[...], preferred_element_type=jnp.float32)
    a_d = jnp.dot(xw, ad0_ref[...], preferred_element_type=jnp.float32)
    xw_ref[...] = xw
    asad_ref[...] = jnp.concatenate([a_s, a_d], axis=1)


def _knn_kernel(post_ref, posa_ref, p2r_ref, p2c_ref, out_ref, *, n):
    pos_t = post_ref[0]
    pos_a = posa_ref[0]
    cross = _tdot(pos_t, pos_a)
    d2 = p2c_ref[0] + p2r_ref[0] - 2.0 * cross
    dist = jnp.sqrt(jnp.maximum(d2, 0.0))
    t = pl.program_id(1)
    rows = t * TILE + lax.broadcasted_iota(jnp.int32, (TILE, 1), 0)
    cols = lax.broadcasted_iota(jnp.int32, (TILE, n), 1)
    self_eq = cols == rows
    dist = jnp.where(self_eq, dist + 1e10, dist)
    mask = self_eq
    for _ in range(K):
        idx = jnp.argmin(dist, axis=1)[:, None]
        eq = cols == idx
        mask = mask | eq
        dist = jnp.where(eq, jnp.float32(jnp.inf), dist)
    out_ref[0] = mask.astype(jnp.int8)


LOG2E = 1.4426950408889634
AST = 40


def _flash_step(xw_ref, as_ref, adb_ref, msk_ref, acc_scr, *, n):
    t = pl.program_id(1)

    @pl.when(t == 0)
    def _init():
        acc_scr[...] = jnp.zeros((AST * H, n), jnp.float32)

    maskf = msk_ref[0].astype(jnp.float32)
    xw = xw_ref[...]
    adb = adb_ref[0]
    asc1 = as_ref[...] * LOG2E
    asc2 = as_ref[...] * (0.2 * LOG2E)
    ones = jnp.ones((xw.shape[0], 1), jnp.float32)
    for h in range(H):
        a1 = asc1[:, h:h + 1] + adb[h:h + 1, :]
        a2 = asc2[:, h:h + 1] + adb[H + h:H + h + 1, :]
        e = jnp.exp2(jnp.maximum(a1, a2)) * maskf
        lhs = jnp.concatenate([xw[:, C * h:C * (h + 1)], ones], axis=1)
        sl = slice(AST * h, AST * h + C + 1)
        acc_scr[sl, :] = acc_scr[sl, :] + _tdot(lhs, e)


def _ln_relu(y, g, b):
    mu = jnp.mean(y, axis=0, keepdims=True)
    var = jnp.mean((y - mu) * (y - mu), axis=0, keepdims=True)
    return jnp.maximum((y - mu) / jnp.sqrt(var + 1e-5) * g + b, 0.0)


def _gat_out(acc_scr, bias_ref, g_ref, bt_ref, *, n):
    y = jnp.concatenate(
        [acc_scr[AST * h:AST * h + C, :]
         / (acc_scr[AST * h + C:AST * h + C + 1, :] + 1e-16)
         for h in range(H)], axis=0) + bias_ref[...]
    return _ln_relu(y, g_ref[...], bt_ref[...])


def _gat0_kernel(xw_ref, as_ref, adb_ref, msk_ref, bias_ref, g_ref, bt_ref,
                 w1_ref, as1_ref, ad1_ref, xw1t_ref, asad1_ref,
                 acc_scr, *, n, nt):
    _flash_step(xw_ref, as_ref, adb_ref, msk_ref, acc_scr, n=n)

    @pl.when(pl.program_id(1) == nt - 1)
    def _fin():
        x1 = _gat_out(acc_scr, bias_ref, g_ref, bt_ref, n=n)
        xw1t = _tdot(w1_ref[...], x1)
        as1t = _tdot(as1_ref[...], xw1t)
        ad1t = _tdot(ad1_ref[...], xw1t)
        asmax = jnp.max(as1t, axis=1, keepdims=True)
        b1t = asmax + ad1t
        b1t = jnp.maximum(b1t, 0.2 * b1t)
        xw1t_ref[0] = xw1t
        asad1_ref[0] = jnp.concatenate(
            [as1t, (ad1t - b1t) * LOG2E, (0.2 * ad1t - b1t) * LOG2E,
             jnp.zeros((4, n), jnp.float32)], axis=0)


def _gat1_kernel(xw_ref, as_ref, adb_ref, msk_ref, bias_ref, g_ref, bt_ref,
                 aw1_ref, ab1_ref, aw2_ref, ab2_ref,
                 cw1_ref, cb1_ref, cw2_ref, cb2_ref, head_ref,
                 acc_scr, *, n, nt):
    _flash_step(xw_ref, as_ref, adb_ref, msk_ref, acc_scr, n=n)

    @pl.when(pl.program_id(1) == nt - 1)
    def _fin():
        x2 = _gat_out(acc_scr, bias_ref, g_ref, bt_ref, n=n)
        h_a = jnp.maximum(_tdot(aw1_ref[...], x2) + ab1_ref[...], 0.0)
        act = jnp.tanh(_tdot(aw2_ref[...], h_a) + ab2_ref[...])
        h_c = jnp.maximum(_tdot(cw1_ref[...], x2) + cb1_ref[...], 0.0)
        val = _tdot(cw2_ref[...], h_c) + cb2_ref[...]
        riota = lax.broadcasted_iota(jnp.int32, (8, n), 0)
        head_ref[0] = jnp.where(riota == 3, val, act)


def _mix(a):
    return (jnp.eye(H, dtype=a.dtype)[:, None, :] * a[:, :, None]).reshape(
        H * C, H)


@jax.jit
def kernel(obs, enc_w, enc_b, enc_g, enc_bt, tem_w, tem_b,
           g0_w, g0_as, g0_ad, g0_b, g0_g, g0_bt,
           g1_w, g1_as, g1_ad, g1_b, g1_g, g1_bt,
           act_w1, act_b1, act_w2, act_b2, cr_w1, cr_b1, cr_w2, cr_b2):
    B, N, D = obs.shape
    Nt = B * N
    NT = N // TILE
    NB = Nt // TILE
    f32 = jnp.float32

    flat = obs.reshape(Nt, D)
    pos_t = jnp.transpose(obs[:, :, :3], (0, 2, 1))
    p2 = jnp.sum(pos_t * pos_t, axis=1)
    p2r = p2[:, None, :]
    p2c = p2[:, :, None]

    row2 = lambda v: v.reshape(1, -1)
    col2 = lambda v: v.reshape(-1, 1)
    full = lambda a: pl.BlockSpec(a.shape, lambda b, t: (0,) * a.ndim)

    As0, Ad0 = _mix(g0_as), _mix(g0_ad)
    xw0, asad0 = pl.pallas_call(
        _enc_kernel,
        grid=(NB,),
        in_specs=[
            pl.BlockSpec((TILE, D), lambda i: (i, 0)),
            pl.BlockSpec((9, HID), lambda i: (0, 0)),
            pl.BlockSpec((1, HID), lambda i: (0, 0)),
            pl.BlockSpec((1, HID), lambda i: (0, 0)),
            pl.BlockSpec((1, HID), lambda i: (0, 0)),
            pl.BlockSpec((1, HID), lambda i: (0, 0)),
            pl.BlockSpec((1, HID), lambda i: (0, 0)),
            pl.BlockSpec((HID, HID), lambda i: (0, 0)),
            pl.BlockSpec((HID, H), lambda i: (0, 0)),
            pl.BlockSpec((HID, H), lambda i: (0, 0)),
        ],
        out_specs=[
            pl.BlockSpec((TILE, HID), lambda i: (i, 0)),
            pl.BlockSpec((TILE, 2 * H), lambda i: (i, 0)),
        ],
        out_shape=[
            jax.ShapeDtypeStruct((Nt, HID), f32),
            jax.ShapeDtypeStruct((Nt, 2 * H), f32),
        ],
    )(flat, enc_w, row2(enc_b), row2(enc_g), row2(enc_bt),
      row2(tem_w), row2(tem_b), g0_w, As0, Ad0)

    adj = pl.pallas_call(
        functools.partial(_knn_kernel, n=N),
        grid=(B, NT),
        in_specs=[
            pl.BlockSpec((1, 3, TILE), lambda b, t: (b, 0, t)),
            pl.BlockSpec((1, 3, N), lambda b, t: (b, 0, 0)),
            pl.BlockSpec((1, 1, N), lambda b, t: (b, 0, 0)),
            pl.BlockSpec((1, TILE, 1), lambda b, t: (b, t, 0)),
        ],
        out_specs=pl.BlockSpec((1, TILE, N), lambda b, t: (b, t, 0)),
        out_shape=jax.ShapeDtypeStruct((B, N, N), jnp.int8),
    )(pos_t, pos_t, p2r, p2c)

    scratch = [
        pltpu.VMEM((AST * H, N), f32),
    ]
    flash_specs = [
        pl.BlockSpec((TILE, HID), lambda b, t: (b * NT + t, 0)),
        pl.BlockSpec((TILE, H), lambda b, t: (b * NT + t, 0)),
        pl.BlockSpec((1, 2 * H, N), lambda b, t: (b, 0, 0)),
        pl.BlockSpec((1, TILE, N), lambda b, t: (b, t, 0)),
    ]

    As1, Ad1 = _mix(g1_as), _mix(g1_ad)
    as0 = asad0[:, :H]
    ad0t = jnp.transpose(asad0[:, H:].reshape(B, N, H), (0, 2, 1))
    b0t = jnp.max(as0.reshape(B, N, H), axis=1)[:, :, None] + ad0t
    b0t = jnp.maximum(b0t, 0.2 * b0t)
    adb0t = jnp.concatenate(
        [(ad0t - b0t) * LOG2E, (0.2 * ad0t - b0t) * LOG2E], axis=1)
    xw1t, asad1t = pl.pallas_call(
        functools.partial(_gat0_kernel, n=N, nt=NT),
        grid=(B, NT),
        in_specs=flash_specs + [
            full(col2(g0_b)), full(col2(g0_g)), full(col2(g0_bt)),
            full(g1_w), full(As1), full(Ad1),
        ],
        out_specs=[
            pl.BlockSpec((1, HID, N), lambda b, t: (b, 0, 0)),
            pl.BlockSpec((1, 4 * H, N), lambda b, t: (b, 0, 0)),
        ],
        out_shape=[
            jax.ShapeDtypeStruct((B, HID, N), f32),
            jax.ShapeDtypeStruct((B, 4 * H, N), f32),
        ],
        scratch_shapes=scratch,
    )(xw0, as0, adb0t, adj, col2(g0_b), col2(g0_g), col2(g0_bt),
      g1_w, As1, Ad1)

    xw1 = jnp.transpose(xw1t, (0, 2, 1)).reshape(Nt, HID)
    as1 = jnp.transpose(asad1t[:, :H, :], (0, 2, 1)).reshape(Nt, H)
    adb1t = asad1t[:, H:3 * H, :]
    aw2 = jnp.pad(act_w2, ((0, 0), (0, 8 - act_w2.shape[1])))
    ab2 = col2(jnp.pad(act_b2, (0, 8 - act_b2.shape[0])))
    cw2 = jnp.pad(cr_w2, ((0, 0), (3, 4)))
    cb2 = col2(jnp.pad(cr_b2, (3, 4)))
    head = pl.pallas_call(
        functools.partial(_gat1_kernel, n=N, nt=NT),
        grid=(B, NT),
        in_specs=flash_specs + [
            full(col2(g1_b)), full(col2(g1_g)), full(col2(g1_bt)),
            full(act_w1), full(col2(act_b1)), full(aw2), full(ab2),
            full(cr_w1), full(col2(cr_b1)), full(cw2), full(cb2),
        ],
        out_specs=pl.BlockSpec((1, 8, N), lambda b, t: (b, 0, 0)),
        out_shape=jax.ShapeDtypeStruct((B, 8, N), f32),
        scratch_shapes=scratch,
    )(xw1, as1, adb1t, adj, col2(g1_b), col2(g1_g), col2(g1_bt),
      act_w1, col2(act_b1), aw2, ab2, cr_w1, col2(cr_b1), cw2, cb2)

    action = jnp.transpose(head[:, :3, :], (0, 2, 1))
    value = jnp.transpose(head[:, 3:4, :], (0, 2, 1))
    return (action, value)
